# Initial kernel scaffold; baseline (speedup 1.0000x reference)
import functools

import jax
import jax.numpy as jnp
from jax import lax
from jax.experimental import pallas as pl
from jax.experimental.pallas import tpu as pltpu
from jax.experimental.pallas import tpu_sc as plsc

N = 10000
E = 320000
D_IN = 128
D_G = 144
D_HID = 256
N_CLASS = 40
D_G2 = 48

NC, NS = 2, 16
NW = NC * NS
EPW = E // NW
K = 80
NCH = EPW // K
ROWS_PT = N // NS

_MESH = plsc.VectorSubcoreMesh(core_axis_name="c", subcore_axis_name="s")


def _fill_const(ref, nrows, ncols, val):
    v = jnp.full((16,), val, jnp.float32)

    def row(i, _):
        for cc in range(ncols // 16):
            ref[i, pl.ds(cc * 16, 16)] = v
        return 0

    lax.fori_loop(0, nrows, row, 0)


@functools.partial(
    pl.kernel,
    out_type=jax.ShapeDtypeStruct((NC, 2, N), jnp.float32),
    mesh=_MESH,
    scratch_types=[
        pltpu.VMEM((NCH, K), jnp.int32),
        pltpu.VMEM((NCH, K), jnp.int32),
        pltpu.VMEM((K,), jnp.float32),
        pltpu.VMEM((1008,), jnp.float32),
        pltpu.VMEM_SHARED((N,), jnp.float32),
        pltpu.VMEM_SHARED((N,), jnp.float32),
    ],
)
def _deg_kernel(src_hbm, dst_hbm, out_hbm, src_v, dst_v, ones_v, zbuf, deg_o, deg_i):
    c = lax.axis_index("c")
    s = lax.axis_index("s")
    w = c * NS + s

    def zfill(i, _):
        zbuf[pl.ds(i * 16, 16)] = jnp.zeros((16,), jnp.float32)
        return 0

    lax.fori_loop(0, 1008 // 16, zfill, 0)
    for i in range(K // 16):
        ones_v[pl.ds(i * 16, 16)] = jnp.ones((16,), jnp.float32)

    @pl.when(s < 10)
    def _():
        pltpu.sync_copy(zbuf.at[pl.ds(0, 1000)], deg_o.at[pl.ds(s * 1000, 1000)])

    @pl.when(s >= 10)
    def _():
        pltpu.sync_copy(zbuf.at[pl.ds(0, 1000)], deg_i.at[pl.ds((s - 10) * 1000, 1000)])

    @pl.when(s < 4)
    def _():
        pltpu.sync_copy(zbuf.at[pl.ds(0, 1000)], deg_i.at[pl.ds((s + 6) * 1000, 1000)])

    pltpu.sync_copy(src_hbm.at[pl.ds(w * NCH, NCH)], src_v)
    pltpu.sync_copy(dst_hbm.at[pl.ds(w * NCH, NCH)], dst_v)
    plsc.subcore_barrier()

    def body(j, _):
        pltpu.sync_copy(ones_v, deg_o.at[src_v.at[j]], add=True)
        pltpu.sync_copy(ones_v, deg_i.at[dst_v.at[j]], add=True)
        return 0

    lax.fori_loop(0, NCH, body, 0)
    plsc.subcore_barrier()

    @pl.when(s < 10)
    def _():
        pltpu.sync_copy(
            deg_o.at[pl.ds(s * 1000, 1000)], out_hbm.at[c, 0, pl.ds(s * 1000, 1000)]
        )

    @pl.when(s >= 10)
    def _():
        pltpu.sync_copy(
            deg_i.at[pl.ds((s - 10) * 1000, 1000)],
            out_hbm.at[c, 1, pl.ds((s - 10) * 1000, 1000)],
        )

    @pl.when(s < 4)
    def _():
        pltpu.sync_copy(
            deg_i.at[pl.ds((s + 6) * 1000, 1000)],
            out_hbm.at[c, 1, pl.ds((s + 6) * 1000, 1000)],
        )


def _make_agg(d):
    @functools.partial(
        pl.kernel,
        out_type=jax.ShapeDtypeStruct((NC, N, d), jnp.float32),
        mesh=_MESH,
        scratch_types=[
            pltpu.VMEM((NCH, K), jnp.int32),
            pltpu.VMEM((NCH, K), jnp.int32),
            pltpu.VMEM((K, d), jnp.float32),
            pltpu.VMEM((25, d), jnp.float32),
            pltpu.VMEM_SHARED((N, d), jnp.float32),
            pltpu.SemaphoreType.DMA,
        ],
    )
    def agg(g_hbm, src_hbm, dst_hbm, out_hbm, src_v, dst_v, buf, zbuf, acc, sem):
        c = lax.axis_index("c")
        s = lax.axis_index("s")
        w = c * NS + s

        _fill_const(zbuf, 25, d, 0.0)

        def zrow(t, _):
            pltpu.sync_copy(zbuf, acc.at[pl.ds(s * ROWS_PT + t * 25, 25)])
            return 0

        lax.fori_loop(0, ROWS_PT // 25, zrow, 0)

        pltpu.sync_copy(src_hbm.at[pl.ds(w * NCH, NCH)], src_v)
        pltpu.sync_copy(dst_hbm.at[pl.ds(w * NCH, NCH)], dst_v)
        plsc.subcore_barrier()

        def body(j, _):
            pltpu.async_copy(g_hbm.at[src_v.at[j]], buf, sem).wait()
            pltpu.sync_copy(buf, acc.at[dst_v.at[j]], add=True)
            return 0

        lax.fori_loop(0, NCH, body, 0)
        plsc.subcore_barrier()

        pltpu.sync_copy(
            acc.at[pl.ds(s * ROWS_PT, ROWS_PT)],
            out_hbm.at[c, pl.ds(s * ROWS_PT, ROWS_PT)],
        )

    return agg


_agg_144 = _make_agg(D_G)
_agg_48 = _make_agg(D_G2)


def _prep_body(degp_ref, x_ref, g_ref, sc_ref):
    deg = degp_ref[0] + degp_ref[1]
    s_out = lax.rsqrt(deg[0] + 1.0)
    s_in = lax.rsqrt(deg[1] + 1.0)
    sc_ref[0, :] = s_out
    sc_ref[1, :] = s_in
    g_ref[:, :D_IN] = x_ref[...] * s_out[:, None]
    tail = jnp.concatenate(
        [s_out[:, None], jnp.zeros((N, D_G - D_IN - 1), jnp.float32)], axis=1
    )
    g_ref[:, D_IN:] = tail


_prep = pl.pallas_call(
    _prep_body,
    out_shape=(
        jax.ShapeDtypeStruct((N, D_G), jnp.float32),
        jax.ShapeDtypeStruct((2, N), jnp.float32),
    ),
)


def _mid_body(p_ref, sc_ref, w1_ref, b1_ref, w2_ref, b2_ref, g2_ref):
    agg = p_ref[0] + p_ref[1]
    s_out = sc_ref[0]
    s_in = sc_ref[1]
    ax = agg[:, :D_IN] * s_in[:, None]
    a1 = agg[:, D_IN] * s_in
    h1 = jnp.dot(ax, w1_ref[...], preferred_element_type=jnp.float32)
    h1 = jnp.maximum(h1 + a1[:, None] * b1_ref[...][None, :], 0.0)
    g2 = jnp.dot(h1, w2_ref[...], preferred_element_type=jnp.float32)
    g2 = (g2 + b2_ref[...][None, :]) * s_out[:, None]
    g2_ref[...] = g2


_mid = pl.pallas_call(
    _mid_body,
    out_shape=jax.ShapeDtypeStruct((N, D_G2), jnp.float32),
)


def _out_body(p_ref, sc_ref, out_ref):
    agg = p_ref[0] + p_ref[1]
    logits = agg[:, :N_CLASS] * sc_ref[1][:, None]
    m = jnp.max(logits, axis=1, keepdims=True)
    ex = jnp.exp(logits - m)
    lse = jnp.log(jnp.sum(ex, axis=1, keepdims=True))
    out_ref[...] = logits - m - lse


_out = pl.pallas_call(
    _out_body,
    out_shape=jax.ShapeDtypeStruct((N, N_CLASS), jnp.float32),
)


def kernel(x, edge_index, W1, b1, W2, b2):
    src = edge_index[0].astype(jnp.int32).reshape(NW * NCH, K)
    dst = edge_index[1].astype(jnp.int32).reshape(NW * NCH, K)
    w2p = jnp.pad(W2, ((0, 0), (0, D_G2 - N_CLASS)))
    b2p = jnp.pad(b2, (0, D_G2 - N_CLASS))

    degp = _deg_kernel(src, dst)
    g, scales = _prep(degp, x)
    p1 = _agg_144(g, src, dst)
    g2 = _mid(p1, scales, W1, b1, w2p, b2p)
    p2 = _agg_48(g2, src, dst)
    return _out(p2, scales)


# speedup vs baseline: 17.4666x; 17.4666x over previous
import functools

import jax
import jax.numpy as jnp
from jax import lax
from jax.experimental import pallas as pl
from jax.experimental.pallas import tpu as pltpu
from jax.experimental.pallas import tpu_sc as plsc

N = 10000
E = 320000
D_IN = 128
D_G = 144
D_HID = 256
N_CLASS = 40
D_G2 = 48

NC, NS = 2, 16
NW = NC * NS
EPW = E // NW
K = 80
NCH = EPW // K
ROWS_PT = N // NS

_MESH = plsc.VectorSubcoreMesh(core_axis_name="c", subcore_axis_name="s")


def _fill_const(ref, nrows, ncols, val):
    v = jnp.full((16,), val, jnp.float32)

    def row(i, _):
        for cc in range(ncols // 16):
            ref[i, pl.ds(cc * 16, 16)] = v
        return 0

    lax.fori_loop(0, nrows, row, 0)


@functools.partial(
    pl.kernel,
    out_type=jax.ShapeDtypeStruct((NC, 2, N), jnp.float32),
    mesh=_MESH,
    compiler_params=pltpu.CompilerParams(use_tc_tiling_on_sc=False),
    scratch_types=[
        pltpu.VMEM((NCH, K), jnp.int32),
        pltpu.VMEM((NCH, K), jnp.int32),
        pltpu.VMEM((K,), jnp.float32),
        pltpu.VMEM((1008,), jnp.float32),
        pltpu.VMEM_SHARED((N,), jnp.float32),
        pltpu.VMEM_SHARED((N,), jnp.float32),
    ],
)
def _deg_kernel(src_hbm, dst_hbm, out_hbm, src_v, dst_v, ones_v, zbuf, deg_o, deg_i):
    c = lax.axis_index("c")
    s = lax.axis_index("s")
    w = c * NS + s

    def zfill(i, _):
        zbuf[pl.ds(i * 16, 16)] = jnp.zeros((16,), jnp.float32)
        return 0

    lax.fori_loop(0, 1008 // 16, zfill, 0)
    for i in range(K // 16):
        ones_v[pl.ds(i * 16, 16)] = jnp.ones((16,), jnp.float32)

    @pl.when(s < 10)
    def _():
        pltpu.sync_copy(zbuf.at[pl.ds(0, 1000)], deg_o.at[pl.ds(s * 1000, 1000)])

    @pl.when(s >= 10)
    def _():
        pltpu.sync_copy(zbuf.at[pl.ds(0, 1000)], deg_i.at[pl.ds((s - 10) * 1000, 1000)])

    @pl.when(s < 4)
    def _():
        pltpu.sync_copy(zbuf.at[pl.ds(0, 1000)], deg_i.at[pl.ds((s + 6) * 1000, 1000)])

    pltpu.sync_copy(src_hbm.at[pl.ds(w * NCH, NCH)], src_v)
    pltpu.sync_copy(dst_hbm.at[pl.ds(w * NCH, NCH)], dst_v)
    plsc.subcore_barrier()

    def body(j, _):
        pltpu.sync_copy(ones_v, deg_o.at[src_v.at[j]], add=True)
        pltpu.sync_copy(ones_v, deg_i.at[dst_v.at[j]], add=True)
        return 0

    lax.fori_loop(0, NCH, body, 0)
    plsc.subcore_barrier()

    @pl.when(s < 10)
    def _():
        pltpu.sync_copy(
            deg_o.at[pl.ds(s * 1000, 1000)], out_hbm.at[c, 0, pl.ds(s * 1000, 1000)]
        )

    @pl.when(s >= 10)
    def _():
        pltpu.sync_copy(
            deg_i.at[pl.ds((s - 10) * 1000, 1000)],
            out_hbm.at[c, 1, pl.ds((s - 10) * 1000, 1000)],
        )

    @pl.when(s < 4)
    def _():
        pltpu.sync_copy(
            deg_i.at[pl.ds((s + 6) * 1000, 1000)],
            out_hbm.at[c, 1, pl.ds((s + 6) * 1000, 1000)],
        )


def _make_agg(d):
    @functools.partial(
        pl.kernel,
        out_type=jax.ShapeDtypeStruct((NC, N, d), jnp.float32),
        mesh=_MESH,
        compiler_params=pltpu.CompilerParams(use_tc_tiling_on_sc=False),
        scratch_types=[
            pltpu.VMEM((NCH, K), jnp.int32),
            pltpu.VMEM((NCH, K), jnp.int32),
            pltpu.VMEM((K, d), jnp.float32),
            pltpu.VMEM((25, d), jnp.float32),
            pltpu.VMEM_SHARED((N, d), jnp.float32),
            pltpu.SemaphoreType.DMA,
        ],
    )
    def agg(g_hbm, src_hbm, dst_hbm, out_hbm, src_v, dst_v, buf, zbuf, acc, sem):
        c = lax.axis_index("c")
        s = lax.axis_index("s")
        w = c * NS + s

        _fill_const(zbuf, 25, d, 0.0)

        def zrow(t, _):
            pltpu.sync_copy(zbuf, acc.at[pl.ds(s * ROWS_PT + t * 25, 25)])
            return 0

        lax.fori_loop(0, ROWS_PT // 25, zrow, 0)

        pltpu.sync_copy(src_hbm.at[pl.ds(w * NCH, NCH)], src_v)
        pltpu.sync_copy(dst_hbm.at[pl.ds(w * NCH, NCH)], dst_v)
        plsc.subcore_barrier()

        def body(j, _):
            pltpu.async_copy(g_hbm.at[src_v.at[j]], buf, sem).wait()
            pltpu.sync_copy(buf, acc.at[dst_v.at[j]], add=True)
            return 0

        lax.fori_loop(0, NCH, body, 0)
        plsc.subcore_barrier()

        pltpu.sync_copy(
            acc.at[pl.ds(s * ROWS_PT, ROWS_PT)],
            out_hbm.at[c, pl.ds(s * ROWS_PT, ROWS_PT)],
        )

    return agg


_agg_144 = _make_agg(D_G)
_agg_48 = _make_agg(D_G2)


def _prep_body(degp_ref, x_ref, g_ref, sc_ref):
    deg = degp_ref[0] + degp_ref[1]
    s_out = lax.rsqrt(deg[0] + 1.0)
    s_in = lax.rsqrt(deg[1] + 1.0)
    sc_ref[0, :] = s_out
    sc_ref[1, :] = s_in
    g_ref[:, :D_IN] = x_ref[...] * s_out[:, None]
    tail = jnp.concatenate(
        [s_out[:, None], jnp.zeros((N, D_G - D_IN - 1), jnp.float32)], axis=1
    )
    g_ref[:, D_IN:] = tail


_prep = pl.pallas_call(
    _prep_body,
    out_shape=(
        jax.ShapeDtypeStruct((N, D_G), jnp.float32),
        jax.ShapeDtypeStruct((2, N), jnp.float32),
    ),
)


def _mid_body(p_ref, sc_ref, w1_ref, b1_ref, w2_ref, b2_ref, g2_ref):
    agg = p_ref[0] + p_ref[1]
    s_out = sc_ref[0]
    s_in = sc_ref[1]
    ax = agg[:, :D_IN] * s_in[:, None]
    a1 = agg[:, D_IN] * s_in
    h1 = jnp.dot(ax, w1_ref[...], preferred_element_type=jnp.float32)
    h1 = jnp.maximum(h1 + a1[:, None] * b1_ref[...][None, :], 0.0)
    g2 = jnp.dot(h1, w2_ref[...], preferred_element_type=jnp.float32)
    g2 = (g2 + b2_ref[...][None, :]) * s_out[:, None]
    g2_ref[...] = g2


_mid = pl.pallas_call(
    _mid_body,
    out_shape=jax.ShapeDtypeStruct((N, D_G2), jnp.float32),
)


def _out_body(p_ref, sc_ref, out_ref):
    agg = p_ref[0] + p_ref[1]
    logits = agg[:, :N_CLASS] * sc_ref[1][:, None]
    m = jnp.max(logits, axis=1, keepdims=True)
    ex = jnp.exp(logits - m)
    lse = jnp.log(jnp.sum(ex, axis=1, keepdims=True))
    out_ref[...] = logits - m - lse


_out = pl.pallas_call(
    _out_body,
    out_shape=jax.ShapeDtypeStruct((N, N_CLASS), jnp.float32),
)


def kernel(x, edge_index, W1, b1, W2, b2):
    src = edge_index[0].astype(jnp.int32).reshape(NW * NCH, K)
    dst = edge_index[1].astype(jnp.int32).reshape(NW * NCH, K)
    w2p = jnp.pad(W2, ((0, 0), (0, D_G2 - N_CLASS)))
    b2p = jnp.pad(b2, (0, D_G2 - N_CLASS))

    degp = _deg_kernel(src, dst)
    g, scales = _prep(degp, x)
    p1 = _agg_144(g, src, dst)
    g2 = _mid(p1, scales, W1, b1, w2p, b2p)
    p2 = _agg_48(g2, src, dst)
    return _out(p2, scales)


# speedup vs baseline: 24.6571x; 1.4117x over previous
import functools

import jax
import jax.numpy as jnp
from jax import lax
from jax.experimental import pallas as pl
from jax.experimental.pallas import tpu as pltpu
from jax.experimental.pallas import tpu_sc as plsc

N = 10000
E = 320000
D_IN = 128
D_G = 144
D_HID = 256
N_CLASS = 40
D_G2 = 48

NC, NS = 2, 16
NW = NC * NS
EPW = E // NW
K = 80
NCH = EPW // K
NBUF = 5
ROWS_PT = N // NS

_MESH = plsc.VectorSubcoreMesh(core_axis_name="c", subcore_axis_name="s")


def _fill_const(ref, nrows, ncols, val):
    v = jnp.full((16,), val, jnp.float32)

    def row(i, _):
        for cc in range(ncols // 16):
            ref[i, pl.ds(cc * 16, 16)] = v
        return 0

    lax.fori_loop(0, nrows, row, 0)


@functools.partial(
    pl.kernel,
    out_type=jax.ShapeDtypeStruct((NC, 2, N), jnp.float32),
    mesh=_MESH,
    compiler_params=pltpu.CompilerParams(use_tc_tiling_on_sc=False),
    scratch_types=[
        pltpu.VMEM((NCH, K), jnp.int32),
        pltpu.VMEM((NCH, K), jnp.int32),
        pltpu.VMEM((K,), jnp.float32),
        pltpu.VMEM((1008,), jnp.float32),
        pltpu.VMEM_SHARED((N,), jnp.float32),
        pltpu.VMEM_SHARED((N,), jnp.float32),
        pltpu.SemaphoreType.DMA,
        pltpu.SemaphoreType.DMA,
    ],
)
def _deg_kernel(
    src_hbm, dst_hbm, out_hbm, src_v, dst_v, ones_v, zbuf, deg_o, deg_i, semo, semi
):
    c = lax.axis_index("c")
    s = lax.axis_index("s")
    w = c * NS + s

    def zfill(i, _):
        zbuf[pl.ds(i * 16, 16)] = jnp.zeros((16,), jnp.float32)
        return 0

    lax.fori_loop(0, 1008 // 16, zfill, 0)
    for i in range(K // 16):
        ones_v[pl.ds(i * 16, 16)] = jnp.ones((16,), jnp.float32)

    @pl.when(s < 10)
    def _():
        pltpu.sync_copy(zbuf.at[pl.ds(0, 1000)], deg_o.at[pl.ds(s * 1000, 1000)])

    @pl.when(s >= 10)
    def _():
        pltpu.sync_copy(zbuf.at[pl.ds(0, 1000)], deg_i.at[pl.ds((s - 10) * 1000, 1000)])

    @pl.when(s < 4)
    def _():
        pltpu.sync_copy(zbuf.at[pl.ds(0, 1000)], deg_i.at[pl.ds((s + 6) * 1000, 1000)])

    pltpu.sync_copy(src_hbm.at[pl.ds(w * NCH, NCH)], src_v)
    pltpu.sync_copy(dst_hbm.at[pl.ds(w * NCH, NCH)], dst_v)
    plsc.subcore_barrier()

    def body(grp, _):
        j0 = grp * NBUF
        puts = []
        for i in range(NBUF):
            puts.append(pltpu.async_copy(ones_v, deg_o.at[src_v.at[j0 + i]], semo, add=True))
            puts.append(pltpu.async_copy(ones_v, deg_i.at[dst_v.at[j0 + i]], semi, add=True))
        for p in puts:
            p.wait()
        return 0

    lax.fori_loop(0, NCH // NBUF, body, 0)
    plsc.subcore_barrier()

    @pl.when(s < 10)
    def _():
        pltpu.sync_copy(
            deg_o.at[pl.ds(s * 1000, 1000)], out_hbm.at[c, 0, pl.ds(s * 1000, 1000)]
        )

    @pl.when(s >= 10)
    def _():
        pltpu.sync_copy(
            deg_i.at[pl.ds((s - 10) * 1000, 1000)],
            out_hbm.at[c, 1, pl.ds((s - 10) * 1000, 1000)],
        )

    @pl.when(s < 4)
    def _():
        pltpu.sync_copy(
            deg_i.at[pl.ds((s + 6) * 1000, 1000)],
            out_hbm.at[c, 1, pl.ds((s + 6) * 1000, 1000)],
        )


def _process_group(g_hbm, acc, bufs, si_row, di_row, semg, sems, nbuf):
    gets = [
        pltpu.async_copy(g_hbm.at[si_row(i)], bufs[i], semg[i]) for i in range(nbuf)
    ]
    puts = []
    for i in range(nbuf):
        gets[i].wait()
        puts.append(pltpu.async_copy(bufs[i], acc.at[di_row(i)], sems, add=True))
    for p in puts:
        p.wait()


def _make_agg(d, k, nbuf, full_idx):
    nch = EPW // k
    ng = nch // nbuf
    sem_types = [pltpu.SemaphoreType.DMA] * (nbuf + 2)
    if full_idx:
        idx_types = [pltpu.VMEM((nch, k), jnp.int32)] * 2
    else:
        idx_types = [pltpu.VMEM((nbuf, k), jnp.int32)] * 4
        sem_types += [pltpu.SemaphoreType.DMA] * 2
        assert ng % 2 == 0

    @functools.partial(
        pl.kernel,
        out_type=jax.ShapeDtypeStruct((NC, N, d), jnp.float32),
        mesh=_MESH,
        compiler_params=pltpu.CompilerParams(use_tc_tiling_on_sc=False),
        scratch_types=[pltpu.VMEM_SHARED((N, d), jnp.float32)]
        + [pltpu.VMEM((k, d), jnp.float32)] * nbuf
        + idx_types
        + sem_types,
    )
    def agg(g_hbm, src_hbm, dst_hbm, zer_hbm, out_hbm, acc, *refs):
        bufs = refs[:nbuf]
        idx = refs[nbuf : nbuf + len(idx_types)]
        sems = refs[nbuf + len(idx_types) :]
        semg = sems[:nbuf]
        semput, semz = sems[nbuf], sems[nbuf + 1]
        c = lax.axis_index("c")
        s = lax.axis_index("s")
        w = c * NS + s
        base = w * nch

        acc_stripe = acc.at[pl.ds(s * ROWS_PT, ROWS_PT)]
        zcp = pltpu.async_copy(zer_hbm, acc_stripe, semz)

        if full_idx:
            src_v, dst_v = idx
            pltpu.sync_copy(src_hbm.at[pl.ds(base, nch)], src_v)
            pltpu.sync_copy(dst_hbm.at[pl.ds(base, nch)], dst_v)
        else:
            si0, di0, si1, di1 = idx
            semi0, semi1 = sems[nbuf + 2], sems[nbuf + 3]
            pltpu.async_copy(src_hbm.at[pl.ds(base, nbuf)], si0, semi0)
            pltpu.async_copy(dst_hbm.at[pl.ds(base, nbuf)], di0, semi0)
            pltpu.async_copy(src_hbm.at[pl.ds(base + nbuf, nbuf)], si1, semi1)
            pltpu.async_copy(dst_hbm.at[pl.ds(base + nbuf, nbuf)], di1, semi1)

        zcp.wait()
        plsc.subcore_barrier()

        if full_idx:

            def body(grp, _):
                j0 = grp * nbuf
                _process_group(
                    g_hbm, acc, bufs,
                    lambda i: src_v.at[j0 + i], lambda i: dst_v.at[j0 + i],
                    semg, semput, nbuf,
                )
                return 0

            lax.fori_loop(0, ng, body, 0)
        else:

            def body(g2, _):
                for half, (si, di, semi) in enumerate(
                    ((si0, di0, semi0), (si1, di1, semi1))
                ):
                    g = 2 * g2 + half
                    pltpu.make_async_copy(src_hbm.at[pl.ds(0, nbuf)], si, semi).wait()
                    pltpu.make_async_copy(dst_hbm.at[pl.ds(0, nbuf)], di, semi).wait()
                    _process_group(
                        g_hbm, acc, bufs,
                        lambda i: si.at[i], lambda i: di.at[i],
                        semg, semput, nbuf,
                    )
                    pf = jnp.minimum(g + 2, ng - 1) * nbuf + base
                    pltpu.async_copy(src_hbm.at[pl.ds(pf, nbuf)], si, semi)
                    pltpu.async_copy(dst_hbm.at[pl.ds(pf, nbuf)], di, semi)
                return 0

            lax.fori_loop(0, ng // 2, body, 0)
            for si, di, semi in ((si0, di0, semi0), (si1, di1, semi1)):
                pltpu.make_async_copy(src_hbm.at[pl.ds(0, nbuf)], si, semi).wait()
                pltpu.make_async_copy(dst_hbm.at[pl.ds(0, nbuf)], di, semi).wait()

        plsc.subcore_barrier()
        pltpu.sync_copy(acc_stripe, out_hbm.at[c, pl.ds(s * ROWS_PT, ROWS_PT)])

    return agg


K1 = 40
_agg_144 = _make_agg(D_G, K1, NBUF, full_idx=False)
_agg_48 = _make_agg(D_G2, K, NBUF, full_idx=True)


def _prep_body(degp_ref, x_ref, g_ref, sc_ref):
    deg = degp_ref[0] + degp_ref[1]
    s_out = lax.rsqrt(deg[0] + 1.0)
    s_in = lax.rsqrt(deg[1] + 1.0)
    sc_ref[0, :] = s_out
    sc_ref[1, :] = s_in
    g_ref[:, :D_IN] = x_ref[...] * s_out[:, None]
    tail = jnp.concatenate(
        [s_out[:, None], jnp.zeros((N, D_G - D_IN - 1), jnp.float32)], axis=1
    )
    g_ref[:, D_IN:] = tail


_prep = pl.pallas_call(
    _prep_body,
    out_shape=(
        jax.ShapeDtypeStruct((N, D_G), jnp.float32),
        jax.ShapeDtypeStruct((2, N), jnp.float32),
    ),
)


def _mid_body(p_ref, sc_ref, w1_ref, b1_ref, w2_ref, b2_ref, g2_ref):
    agg = p_ref[0] + p_ref[1]
    s_out = sc_ref[0]
    s_in = sc_ref[1]
    ax = agg[:, :D_IN] * s_in[:, None]
    a1 = agg[:, D_IN] * s_in
    h1 = jnp.dot(ax, w1_ref[...], preferred_element_type=jnp.float32)
    h1 = jnp.maximum(h1 + a1[:, None] * b1_ref[...][None, :], 0.0)
    g2 = jnp.dot(h1, w2_ref[...], preferred_element_type=jnp.float32)
    g2 = (g2 + b2_ref[...][None, :]) * s_out[:, None]
    g2_ref[...] = g2


_mid = pl.pallas_call(
    _mid_body,
    out_shape=jax.ShapeDtypeStruct((N, D_G2), jnp.float32),
)


def _out_body(p_ref, sc_ref, out_ref):
    agg = p_ref[0] + p_ref[1]
    logits = agg[:, :N_CLASS] * sc_ref[1][:, None]
    m = jnp.max(logits, axis=1, keepdims=True)
    ex = jnp.exp(logits - m)
    lse = jnp.log(jnp.sum(ex, axis=1, keepdims=True))
    out_ref[...] = logits - m - lse


_out = pl.pallas_call(
    _out_body,
    out_shape=jax.ShapeDtypeStruct((N, N_CLASS), jnp.float32),
)


def kernel(x, edge_index, W1, b1, W2, b2):
    src = edge_index[0].astype(jnp.int32)
    dst = edge_index[1].astype(jnp.int32)
    src80 = src.reshape(NW * NCH, K)
    dst80 = dst.reshape(NW * NCH, K)
    src40 = src.reshape(E // K1, K1)
    dst40 = dst.reshape(E // K1, K1)
    z144 = jnp.zeros((ROWS_PT, D_G), jnp.float32)
    z48 = jnp.zeros((ROWS_PT, D_G2), jnp.float32)
    w2p = jnp.pad(W2, ((0, 0), (0, D_G2 - N_CLASS)))
    b2p = jnp.pad(b2, (0, D_G2 - N_CLASS))

    degp = _deg_kernel(src80, dst80)
    g, scales = _prep(degp, x)
    p1 = _agg_144(g, src40, dst40, z144)
    g2 = _mid(p1, scales, W1, b1, w2p, b2p)
    p2 = _agg_48(g2, src80, dst80, z48)
    return _out(p2, scales)
